# baseline (speedup 1.0000x reference)
import functools
import math

import jax
import jax.numpy as jnp
from jax import lax
from jax.experimental import pallas as pl
from jax.experimental.pallas import tpu as pltpu
from jax.experimental.pallas import tpu_sc as plsc

D = 32
SCALE = math.sqrt(float(D))


def _gather_kernel(V, N, NC, NW):
    C = 512
    CHUNKS = N // (NW * C)
    assert CHUNKS * NW * C == N
    mesh = plsc.VectorSubcoreMesh(core_axis_name="c", subcore_axis_name="s")

    @functools.partial(
        pl.kernel,
        mesh=mesh,
        out_type=jax.ShapeDtypeStruct((N, D), jnp.float32),
        scratch_types=[
            pltpu.VMEM((C,), jnp.int32),
            pltpu.VMEM((C, D), jnp.float32),
            pltpu.SemaphoreType.DMA,
        ],
        compiler_params=pltpu.CompilerParams(use_tc_tiling_on_sc=False),
    )
    def k(idx_hbm, tab_hbm, out_hbm, idx_v, rows_v, sem):
        wid = lax.axis_index("s") * NC + lax.axis_index("c")

        def chunk(g, carry):
            base = (wid * CHUNKS + g) * C
            pltpu.sync_copy(idx_hbm.at[pl.ds(base, C)], idx_v)
            pltpu.async_copy(tab_hbm.at[idx_v], rows_v, sem).wait()

            def row(r, c):
                lo = rows_v[r, pl.ds(0, 16)]
                hi = rows_v[r, pl.ds(16, 16)]
                rows_v[r, pl.ds(0, 16)] = lo * SCALE
                rows_v[r, pl.ds(16, 16)] = hi * SCALE
                return c

            lax.fori_loop(0, C, row, 0, unroll=8)
            pltpu.sync_copy(rows_v, out_hbm.at[pl.ds(base, C), :])
            return carry

        lax.fori_loop(0, CHUNKS, chunk, 0)

    return k


def kernel(x, table):
    B0, B1 = x.shape
    V, d = table.shape
    N = B0 * B1
    idx = x.reshape(N).astype(jnp.int32)
    info = plsc.get_sparse_core_info()
    NC = info.num_cores
    NW = NC * info.num_subcores
    out = _gather_kernel(V, N, NC, NW)(idx, table)
    return out.reshape(B0, B1, d)


# speedup vs baseline: 1.5195x; 1.5195x over previous
import functools
import math

import jax
import jax.numpy as jnp
from jax import lax
from jax.experimental import pallas as pl
from jax.experimental.pallas import tpu as pltpu
from jax.experimental.pallas import tpu_sc as plsc

D = 32
SCALE = math.sqrt(float(D))


def _gather_kernel(V, J, I, NC, NW):
    C = 512
    BPW = C // 128
    TPA = (I // 128) * 8 * 128
    assert NW * C == I
    mesh = plsc.VectorSubcoreMesh(core_axis_name="c", subcore_axis_name="s")

    @functools.partial(
        pl.kernel,
        mesh=mesh,
        out_type=jax.ShapeDtypeStruct((J * D * I,), jnp.float32),
        scratch_types=[
            pltpu.VMEM((C,), jnp.int32),
            pltpu.VMEM((C, D), jnp.float32),
            pltpu.VMEM((D * C,), jnp.float32),
            pltpu.SemaphoreType.DMA,
        ],
        compiler_params=pltpu.CompilerParams(
            use_tc_tiling_on_sc=False, needs_layout_passes=False),
    )
    def k(idx_hbm, tab_hbm, out_hbm, idx_v, rows_v, stage_v, sem):
        wid = lax.axis_index("s") * NC + lax.axis_index("c")
        iota = lax.iota(jnp.int32, 16)
        pos_lo = (iota // 8) * (BPW * 1024) + (iota % 8) * 128
        pos_hi = ((iota + 16) // 8) * (BPW * 1024) + (iota % 8) * 128

        def jloop(j, carry):
            base = j * I + wid * C
            pltpu.sync_copy(idx_hbm.at[pl.ds(base, C)], idx_v)
            pltpu.async_copy(tab_hbm.at[idx_v], rows_v, sem).wait()

            def row(r, c):
                lo = rows_v[r, pl.ds(0, 16)]
                hi = rows_v[r, pl.ds(16, 16)]
                roff = (r // 128) * 1024 + (r % 128)
                plsc.store_scatter(stage_v, [pos_lo + roff], lo * SCALE)
                plsc.store_scatter(stage_v, [pos_hi + roff], hi * SCALE)
                return c

            lax.fori_loop(0, C, row, 0, unroll=8)

            for a in range(D // 8):
                pltpu.sync_copy(
                    stage_v.at[pl.ds(a * BPW * 1024, BPW * 1024)],
                    out_hbm.at[pl.ds(j * D * I + a * TPA + wid * BPW * 1024,
                                     BPW * 1024)])
            return carry

        lax.fori_loop(0, J, jloop, 0)

    return k


def kernel(x, table):
    B0, B1 = x.shape
    V, d = table.shape
    I, J = B0, B1
    idx = x.T.reshape(J * I).astype(jnp.int32)
    info = plsc.get_sparse_core_info()
    NC = info.num_cores
    NW = NC * info.num_subcores
    flat = _gather_kernel(V, J, I, NC, NW)(idx, table)
    out5 = flat.reshape(J, d // 8, I // 128, 8, 128)
    return out5.transpose(2, 4, 0, 1, 3).reshape(I, J, d)
